# Initial kernel scaffold; baseline (speedup 1.0000x reference)
import jax
import jax.numpy as jnp
from jax.experimental import pallas as pl


def kernel(x, edge_index, W1, b1, Wmu, bmu, Wls, bls):
    raise NotImplementedError("write your pallas kernel here")


# speedup vs baseline: 6.0148x; 6.0148x over previous
import functools

import jax
import jax.numpy as jnp
from jax import lax
from jax.experimental import pallas as pl
from jax.experimental.pallas import tpu as pltpu
from jax.experimental.pallas import tpu_sc as plsc

N = 10000
NP = 10240
E = 160000
NW = 32
CH = 128
SLABS = 40
EP = NW * SLABS * CH
ROWS_PER_TILE = NP // 16
BM = 512


def _mesh():
    return plsc.VectorSubcoreMesh(core_axis_name="c", subcore_axis_name="s")


def _deg_body(dst_hbm, ones_hbm, zeros_hbm, out_hbm, idx_v, ones_v, acc_sh):
    c = lax.axis_index("c")
    s = lax.axis_index("s")
    w = c * 16 + s
    base = s * ROWS_PER_TILE
    pltpu.sync_copy(zeros_hbm, acc_sh.at[pl.ds(base, ROWS_PER_TILE)])
    pltpu.sync_copy(dst_hbm.at[w], idx_v)
    pltpu.sync_copy(ones_hbm, ones_v)
    plsc.subcore_barrier()

    def body(j, carry):
        pltpu.sync_copy(ones_v, acc_sh.at[idx_v.at[j]], add=True)
        return carry

    lax.fori_loop(0, SLABS, body, 0)
    plsc.subcore_barrier()
    pltpu.sync_copy(acc_sh.at[pl.ds(base, ROWS_PER_TILE)],
                    out_hbm.at[c, pl.ds(base, ROWS_PER_TILE)])


def _deg_call(dst_r, ones128, zeros128):
    k = functools.partial(
        pl.kernel,
        mesh=_mesh(),
        out_type=jax.ShapeDtypeStruct((2, NP, 128), jnp.float32),
        scratch_types=[
            pltpu.VMEM((SLABS, CH), jnp.int32),
            pltpu.VMEM((CH, 128), jnp.float32),
            pltpu.VMEM_SHARED((NP, 128), jnp.float32),
        ],
    )(_deg_body)
    return k(dst_r, ones128, zeros128)


def _prop_body(g_hbm, src_hbm, dst_hbm, zeros_hbm, out_hbm,
               sidx_v, didx_v, rows_v, sem, acc_sh):
    c = lax.axis_index("c")
    s = lax.axis_index("s")
    w = c * 16 + s
    base = s * ROWS_PER_TILE
    pltpu.sync_copy(zeros_hbm, acc_sh.at[pl.ds(base, ROWS_PER_TILE)])
    pltpu.sync_copy(src_hbm.at[w], sidx_v)
    pltpu.sync_copy(dst_hbm.at[w], didx_v)
    plsc.subcore_barrier()

    def body(j, carry):
        pltpu.async_copy(g_hbm.at[sidx_v.at[j]], rows_v, sem).wait()
        pltpu.sync_copy(rows_v, acc_sh.at[didx_v.at[j]], add=True)
        return carry

    lax.fori_loop(0, SLABS, body, 0)
    plsc.subcore_barrier()
    pltpu.sync_copy(acc_sh.at[pl.ds(base, ROWS_PER_TILE)],
                    out_hbm.at[c, pl.ds(base, ROWS_PER_TILE)])


def _prop_call(g, src_r, dst_r, zeros128):
    k = functools.partial(
        pl.kernel,
        mesh=_mesh(),
        out_type=jax.ShapeDtypeStruct((2, NP, 128), jnp.float32),
        scratch_types=[
            pltpu.VMEM((SLABS, CH), jnp.int32),
            pltpu.VMEM((SLABS, CH), jnp.int32),
            pltpu.VMEM((CH, 128), jnp.float32),
            pltpu.SemaphoreType.DMA,
            pltpu.VMEM_SHARED((NP, 128), jnp.float32),
        ],
    )(_prop_body)
    return k(g, src_r, dst_r, zeros128)


def _enc_a_body(x_ref, w_ref, degp_ref, g1_ref, dinv_ref):
    deg = degp_ref[0, :, :16] + degp_ref[1, :, :16] + 1.0
    dinv = lax.rsqrt(deg)
    dinv_ref[...] = dinv
    h = jnp.dot(x_ref[...], w_ref[...], preferred_element_type=jnp.float32)
    g1_ref[...] = h * dinv[:, 0:1]


def _enc_a_call(x_pad, W1, degp):
    return pl.pallas_call(
        _enc_a_body,
        grid=(NP // BM,),
        in_specs=[
            pl.BlockSpec((BM, 128), lambda i: (i, 0)),
            pl.BlockSpec((128, 128), lambda i: (0, 0)),
            pl.BlockSpec((2, BM, 128), lambda i: (0, i, 0)),
        ],
        out_specs=[
            pl.BlockSpec((BM, 128), lambda i: (i, 0)),
            pl.BlockSpec((BM, 16), lambda i: (i, 0)),
        ],
        out_shape=[
            jax.ShapeDtypeStruct((NP, 128), jnp.float32),
            jax.ShapeDtypeStruct((NP, 16), jnp.float32),
        ],
    )(x_pad, W1, degp)


def _enc_b_body(s1p_ref, g1_ref, dinv_ref, b1_ref, wc_ref, g2_ref):
    dinv = dinv_ref[...][:, 0:1]
    h = s1p_ref[0] + s1p_ref[1] + g1_ref[...]
    h = jnp.maximum(h * dinv + b1_ref[...], 0.0)
    c2 = jnp.dot(h, wc_ref[...], preferred_element_type=jnp.float32)
    g2_ref[...] = c2 * dinv


def _enc_b_call(s1p, g1, dinv16, b1r, Wc):
    return pl.pallas_call(
        _enc_b_body,
        grid=(NP // BM,),
        in_specs=[
            pl.BlockSpec((2, BM, 128), lambda i: (0, i, 0)),
            pl.BlockSpec((BM, 128), lambda i: (i, 0)),
            pl.BlockSpec((BM, 16), lambda i: (i, 0)),
            pl.BlockSpec((1, 128), lambda i: (0, 0)),
            pl.BlockSpec((128, 128), lambda i: (0, 0)),
        ],
        out_specs=pl.BlockSpec((BM, 128), lambda i: (i, 0)),
        out_shape=jax.ShapeDtypeStruct((NP, 128), jnp.float32),
    )(s1p, g1, dinv16, b1r, Wc)


def _enc_c_body(s2p_ref, g2_ref, dinv_ref, bc_ref, out_ref):
    dinv = dinv_ref[...][:, 0:1]
    t = s2p_ref[0] + s2p_ref[1] + g2_ref[...]
    out_ref[...] = t * dinv + bc_ref[...]


def _enc_c_call(s2p, g2, dinv16, bcr):
    return pl.pallas_call(
        _enc_c_body,
        grid=(NP // BM,),
        in_specs=[
            pl.BlockSpec((2, BM, 128), lambda i: (0, i, 0)),
            pl.BlockSpec((BM, 128), lambda i: (i, 0)),
            pl.BlockSpec((BM, 16), lambda i: (i, 0)),
            pl.BlockSpec((1, 128), lambda i: (0, 0)),
        ],
        out_specs=pl.BlockSpec((BM, 128), lambda i: (i, 0)),
        out_shape=jax.ShapeDtypeStruct((NP, 128), jnp.float32),
    )(s2p, g2, dinv16, bcr)


DM = 512
DN = 512


def _dec_body(a_ref, b_ref, o_ref):
    acc = lax.dot_general(a_ref[...], b_ref[...],
                          (((1,), (1,)), ((), ())),
                          preferred_element_type=jnp.float32)
    o_ref[...] = jax.nn.sigmoid(acc)


def _dec_call(z64):
    return pl.pallas_call(
        _dec_body,
        grid=(NP // DM, NP // DN),
        in_specs=[
            pl.BlockSpec((DM, 64), lambda i, j: (i, 0)),
            pl.BlockSpec((DN, 64), lambda i, j: (j, 0)),
        ],
        out_specs=pl.BlockSpec((DM, DN), lambda i, j: (i, j)),
        out_shape=jax.ShapeDtypeStruct((N, N), jnp.float32),
    )(z64, z64)


def kernel(x, edge_index, W1, b1, Wmu, bmu, Wls, bls):
    src = edge_index[0].astype(jnp.int32)
    dst = edge_index[1].astype(jnp.int32)
    pad = jnp.full((EP - E,), N, jnp.int32)
    src_r = jnp.concatenate([src, pad]).reshape(NW, SLABS, CH)
    dst_r = jnp.concatenate([dst, pad]).reshape(NW, SLABS, CH)
    x_pad = jnp.pad(x, ((0, NP - N), (0, 0)))
    Wc = jnp.concatenate([Wmu, Wls], axis=1)
    bc = jnp.concatenate([bmu, bls]).reshape(1, 128)
    b1r = b1.reshape(1, 128)
    ones128 = jnp.ones((CH, 128), jnp.float32)
    zeros128 = jnp.zeros((ROWS_PER_TILE, 128), jnp.float32)

    degp = _deg_call(dst_r, ones128, zeros128)
    g1, dinv16 = _enc_a_call(x_pad, W1, degp)
    s1p = _prop_call(g1, src_r, dst_r, zeros128)
    g2 = _enc_b_call(s1p, g1, dinv16, b1r, Wc)
    s2p = _prop_call(g2, src_r, dst_r, zeros128)
    full2 = _enc_c_call(s2p, g2, dinv16, bc)

    mu = full2[:N, :64]
    logstd = full2[:N, 64:]
    adj = _dec_call(full2[:, :64])
    return adj, mu, logstd


# speedup vs baseline: 6.1226x; 1.0179x over previous
import functools

import jax
import jax.numpy as jnp
from jax import lax
from jax.experimental import pallas as pl
from jax.experimental.pallas import tpu as pltpu
from jax.experimental.pallas import tpu_sc as plsc

N = 10000
NP = 10240
E = 160000
NW = 32
CH = 128
SLABS = 40
EP = NW * SLABS * CH
ROWS_PER_TILE = NP // 16
BM = 512


def _mesh():
    return plsc.VectorSubcoreMesh(core_axis_name="c", subcore_axis_name="s")


def _deg_body(dst_hbm, ones_hbm, zeros_hbm, out_hbm, idx_v, ones_v, dsem,
              acc_sh):
    c = lax.axis_index("c")
    s = lax.axis_index("s")
    w = c * 16 + s
    base = s * ROWS_PER_TILE
    pltpu.sync_copy(zeros_hbm, acc_sh.at[pl.ds(base, ROWS_PER_TILE)])
    pltpu.sync_copy(dst_hbm.at[w], idx_v)
    pltpu.sync_copy(ones_hbm, ones_v)
    plsc.subcore_barrier()

    def body(g, carry):
        descs = [
            pltpu.async_copy(ones_v, acc_sh.at[idx_v.at[g * 8 + b]],
                             dsem, add=True)
            for b in range(8)
        ]
        for d in descs:
            d.wait()
        return carry

    lax.fori_loop(0, SLABS // 8, body, 0)
    plsc.subcore_barrier()
    pltpu.sync_copy(acc_sh.at[pl.ds(base, ROWS_PER_TILE)],
                    out_hbm.at[c, pl.ds(base, ROWS_PER_TILE)])


def _deg_call(dst_r, ones128, zeros128):
    k = functools.partial(
        pl.kernel,
        mesh=_mesh(),
        out_type=jax.ShapeDtypeStruct((2, NP, 128), jnp.float32),
        scratch_types=[
            pltpu.VMEM((SLABS, CH), jnp.int32),
            pltpu.VMEM((CH, 128), jnp.float32),
            pltpu.SemaphoreType.DMA,
            pltpu.VMEM_SHARED((NP, 128), jnp.float32),
        ],
    )(_deg_body)
    return k(dst_r, ones128, zeros128)


NBUF = 2


def _prop_body(g_hbm, src_hbm, dst_hbm, zeros_hbm, out_hbm,
               sidx_v, didx_v, rows_v, gsem, ssem, acc_sh):
    c = lax.axis_index("c")
    s = lax.axis_index("s")
    w = c * 16 + s
    base = s * ROWS_PER_TILE
    pltpu.sync_copy(zeros_hbm, acc_sh.at[pl.ds(base, ROWS_PER_TILE)])
    pltpu.sync_copy(src_hbm.at[w], sidx_v)
    pltpu.sync_copy(dst_hbm.at[w], didx_v)
    plsc.subcore_barrier()

    def body(g, carry):
        gd = [
            pltpu.async_copy(g_hbm.at[sidx_v.at[g * NBUF + b]],
                             rows_v.at[b], gsem.at[b])
            for b in range(NBUF)
        ]
        sd = []
        for b in range(NBUF):
            gd[b].wait()
            sd.append(pltpu.async_copy(rows_v.at[b],
                                       acc_sh.at[didx_v.at[g * NBUF + b]],
                                       ssem.at[b], add=True))
        for d in sd:
            d.wait()
        return carry

    lax.fori_loop(0, SLABS // NBUF, body, 0)
    plsc.subcore_barrier()
    pltpu.sync_copy(acc_sh.at[pl.ds(base, ROWS_PER_TILE)],
                    out_hbm.at[c, pl.ds(base, ROWS_PER_TILE)])


def _prop_call(g, src_r, dst_r, zeros128):
    k = functools.partial(
        pl.kernel,
        mesh=_mesh(),
        out_type=jax.ShapeDtypeStruct((2, NP, 128), jnp.float32),
        scratch_types=[
            pltpu.VMEM((SLABS, CH), jnp.int32),
            pltpu.VMEM((SLABS, CH), jnp.int32),
            pltpu.VMEM((NBUF, CH, 128), jnp.float32),
            pltpu.SemaphoreType.DMA((NBUF,)),
            pltpu.SemaphoreType.DMA((NBUF,)),
            pltpu.VMEM_SHARED((NP, 128), jnp.float32),
        ],
    )(_prop_body)
    return k(g, src_r, dst_r, zeros128)


def _mm_body(x_ref, w_ref, o_ref):
    o_ref[...] = jnp.dot(x_ref[...], w_ref[...],
                         preferred_element_type=jnp.float32)


def _mm_call(x_pad, W1):
    return pl.pallas_call(
        _mm_body,
        grid=(NP // BM,),
        in_specs=[
            pl.BlockSpec((BM, 128), lambda i: (i, 0)),
            pl.BlockSpec((128, 128), lambda i: (0, 0)),
        ],
        out_specs=pl.BlockSpec((BM, 128), lambda i: (i, 0)),
        out_shape=jax.ShapeDtypeStruct((NP, 128), jnp.float32),
    )(x_pad, W1)


def _enc_a_body(h1_ref, degp_ref, g1_ref, dinv_ref):
    deg = degp_ref[0, :, :16] + degp_ref[1, :, :16] + 1.0
    dinv = lax.rsqrt(deg)
    dinv_ref[...] = dinv
    g1_ref[...] = h1_ref[...] * dinv[:, 0:1]


def _enc_a_call(h1, degp):
    return pl.pallas_call(
        _enc_a_body,
        grid=(NP // BM,),
        in_specs=[
            pl.BlockSpec((BM, 128), lambda i: (i, 0)),
            pl.BlockSpec((2, BM, 128), lambda i: (0, i, 0)),
        ],
        out_specs=[
            pl.BlockSpec((BM, 128), lambda i: (i, 0)),
            pl.BlockSpec((BM, 16), lambda i: (i, 0)),
        ],
        out_shape=[
            jax.ShapeDtypeStruct((NP, 128), jnp.float32),
            jax.ShapeDtypeStruct((NP, 16), jnp.float32),
        ],
    )(h1, degp)


def _enc_b_body(s1p_ref, g1_ref, dinv_ref, b1_ref, wc_ref, g2_ref):
    dinv = dinv_ref[...][:, 0:1]
    h = s1p_ref[0] + s1p_ref[1] + g1_ref[...]
    h = jnp.maximum(h * dinv + b1_ref[...], 0.0)
    c2 = jnp.dot(h, wc_ref[...], preferred_element_type=jnp.float32)
    g2_ref[...] = c2 * dinv


def _enc_b_call(s1p, g1, dinv16, b1r, Wc):
    return pl.pallas_call(
        _enc_b_body,
        grid=(NP // BM,),
        in_specs=[
            pl.BlockSpec((2, BM, 128), lambda i: (0, i, 0)),
            pl.BlockSpec((BM, 128), lambda i: (i, 0)),
            pl.BlockSpec((BM, 16), lambda i: (i, 0)),
            pl.BlockSpec((1, 128), lambda i: (0, 0)),
            pl.BlockSpec((128, 128), lambda i: (0, 0)),
        ],
        out_specs=pl.BlockSpec((BM, 128), lambda i: (i, 0)),
        out_shape=jax.ShapeDtypeStruct((NP, 128), jnp.float32),
    )(s1p, g1, dinv16, b1r, Wc)


def _enc_c_body(s2p_ref, g2_ref, dinv_ref, bc_ref, out_ref):
    dinv = dinv_ref[...][:, 0:1]
    t = s2p_ref[0] + s2p_ref[1] + g2_ref[...]
    out_ref[...] = t * dinv + bc_ref[...]


def _enc_c_call(s2p, g2, dinv16, bcr):
    return pl.pallas_call(
        _enc_c_body,
        grid=(NP // BM,),
        in_specs=[
            pl.BlockSpec((2, BM, 128), lambda i: (0, i, 0)),
            pl.BlockSpec((BM, 128), lambda i: (i, 0)),
            pl.BlockSpec((BM, 16), lambda i: (i, 0)),
            pl.BlockSpec((1, 128), lambda i: (0, 0)),
        ],
        out_specs=pl.BlockSpec((BM, 128), lambda i: (i, 0)),
        out_shape=jax.ShapeDtypeStruct((NP, 128), jnp.float32),
    )(s2p, g2, dinv16, bcr)


DM = 512
DN = 512


def _dec_body(a_ref, b_ref, o_ref):
    acc = lax.dot_general(a_ref[...], b_ref[...],
                          (((1,), (1,)), ((), ())),
                          preferred_element_type=jnp.float32)
    o_ref[...] = jax.nn.sigmoid(acc)


def _dec_call(z64):
    return pl.pallas_call(
        _dec_body,
        grid=(NP // DM, NP // DN),
        in_specs=[
            pl.BlockSpec((DM, 64), lambda i, j: (i, 0)),
            pl.BlockSpec((DN, 64), lambda i, j: (j, 0)),
        ],
        out_specs=pl.BlockSpec((DM, DN), lambda i, j: (i, j)),
        out_shape=jax.ShapeDtypeStruct((N, N), jnp.float32),
    )(z64, z64)


def kernel(x, edge_index, W1, b1, Wmu, bmu, Wls, bls):
    src = edge_index[0].astype(jnp.int32)
    dst = edge_index[1].astype(jnp.int32)
    pad = jnp.full((EP - E,), N, jnp.int32)
    src_r = jnp.concatenate([src, pad]).reshape(NW, SLABS, CH)
    dst_r = jnp.concatenate([dst, pad]).reshape(NW, SLABS, CH)
    x_pad = jnp.pad(x, ((0, NP - N), (0, 0)))
    Wc = jnp.concatenate([Wmu, Wls], axis=1)
    bc = jnp.concatenate([bmu, bls]).reshape(1, 128)
    b1r = b1.reshape(1, 128)
    ones128 = jnp.ones((CH, 128), jnp.float32)
    zeros128 = jnp.zeros((ROWS_PER_TILE, 128), jnp.float32)

    degp = _deg_call(dst_r, ones128, zeros128)
    h1 = _mm_call(x_pad, W1)
    g1, dinv16 = _enc_a_call(h1, degp)
    s1p = _prop_call(g1, src_r, dst_r, zeros128)
    g2 = _enc_b_call(s1p, g1, dinv16, b1r, Wc)
    s2p = _prop_call(g2, src_r, dst_r, zeros128)
    full2 = _enc_c_call(s2p, g2, dinv16, bc)

    mu = full2[:N, :64]
    logstd = full2[:N, 64:]
    adj = _dec_call(full2[:, :64])
    return adj, mu, logstd


# speedup vs baseline: 6.7295x; 1.0991x over previous
import functools

import jax
import jax.numpy as jnp
from jax import lax
from jax.experimental import pallas as pl
from jax.experimental.pallas import tpu as pltpu
from jax.experimental.pallas import tpu_sc as plsc

N = 10000
NP = 10240
E = 160000
NW = 32
CH = 128
SLABS = 40
EP = NW * SLABS * CH
ROWS_PER_TILE = NP // 16
BM = 512


def _mesh():
    return plsc.VectorSubcoreMesh(core_axis_name="c", subcore_axis_name="s")


def _deg_body(dst_hbm, ones_hbm, zeros_hbm, out_hbm, idx_v, ones_v, dsem,
              acc_sh):
    c = lax.axis_index("c")
    s = lax.axis_index("s")
    w = c * 16 + s
    base = s * ROWS_PER_TILE
    pltpu.sync_copy(zeros_hbm, acc_sh.at[pl.ds(base, ROWS_PER_TILE)])
    pltpu.sync_copy(dst_hbm.at[w], idx_v)
    pltpu.sync_copy(ones_hbm, ones_v)
    plsc.subcore_barrier()

    def body(g, carry):
        descs = [
            pltpu.async_copy(ones_v, acc_sh.at[idx_v.at[g * 8 + b]],
                             dsem, add=True)
            for b in range(8)
        ]
        for d in descs:
            d.wait()
        return carry

    lax.fori_loop(0, SLABS // 8, body, 0)
    plsc.subcore_barrier()
    pltpu.sync_copy(acc_sh.at[pl.ds(base, ROWS_PER_TILE)],
                    out_hbm.at[c, pl.ds(base, ROWS_PER_TILE)])


def _deg_call(dst_r, ones128, zeros128):
    k = functools.partial(
        pl.kernel,
        mesh=_mesh(),
        out_type=jax.ShapeDtypeStruct((2, NP, 128), jnp.float32),
        scratch_types=[
            pltpu.VMEM((SLABS, CH), jnp.int32),
            pltpu.VMEM((CH, 128), jnp.float32),
            pltpu.SemaphoreType.DMA,
            pltpu.VMEM_SHARED((NP, 128), jnp.float32),
        ],
    )(_deg_body)
    return k(dst_r, ones128, zeros128)


NBUF = 2
NCHUNK = EP // CH
SA = 64
SB = 80 - SA
FAST_C = 0
ACC_RPT = 632
ACC_ROWS = 16 * ACC_RPT


def _prop_body(g_hbm, src_hbm, dst_hbm, zeros_hbm, out_hbm,
               sidx_v, didx_v, rows_v, gsem, ssem, acc_sh):
    c = lax.axis_index("c")
    s = lax.axis_index("s")
    base = s * ACC_RPT
    pltpu.sync_copy(zeros_hbm, acc_sh.at[pl.ds(base, ACC_RPT)])

    def run(start, n_chunks):
        pltpu.sync_copy(src_hbm.at[pl.ds(start, n_chunks)],
                        sidx_v.at[pl.ds(0, n_chunks)])
        pltpu.sync_copy(dst_hbm.at[pl.ds(start, n_chunks)],
                        didx_v.at[pl.ds(0, n_chunks)])
        plsc.subcore_barrier()

        def body(g, carry):
            gd = [
                pltpu.async_copy(g_hbm.at[sidx_v.at[g * NBUF + b]],
                                 rows_v.at[b], gsem.at[b])
                for b in range(NBUF)
            ]
            sd = []
            for b in range(NBUF):
                gd[b].wait()
                sd.append(pltpu.async_copy(
                    rows_v.at[b], acc_sh.at[didx_v.at[g * NBUF + b]],
                    ssem.at[b], add=True))
            for d in sd:
                d.wait()
            return carry

        lax.fori_loop(0, n_chunks // NBUF, body, 0)

    @pl.when(c == FAST_C)
    def _():
        run(s * SA, SA)

    @pl.when(c != FAST_C)
    def _():
        run(16 * SA + s * SB, SB)

    plsc.subcore_barrier()
    pltpu.sync_copy(acc_sh.at[pl.ds(base, ACC_RPT)],
                    out_hbm.at[c, pl.ds(base, ACC_RPT)])


def _prop_call(g, src_f, dst_f, zeros128):
    k = functools.partial(
        pl.kernel,
        mesh=_mesh(),
        out_type=jax.ShapeDtypeStruct((2, ACC_ROWS, 128), jnp.float32),
        scratch_types=[
            pltpu.VMEM((SA, CH), jnp.int32),
            pltpu.VMEM((SA, CH), jnp.int32),
            pltpu.VMEM((NBUF, CH, 128), jnp.float32),
            pltpu.SemaphoreType.DMA((NBUF,)),
            pltpu.SemaphoreType.DMA((NBUF,)),
            pltpu.VMEM_SHARED((ACC_ROWS, 128), jnp.float32),
        ],
    )(_prop_body)
    return k(g, src_f, dst_f, zeros128)


def _mm_body(x_ref, w_ref, o_ref):
    o_ref[...] = jnp.dot(x_ref[...], w_ref[...],
                         preferred_element_type=jnp.float32)


def _mm_call(x_pad, W1):
    return pl.pallas_call(
        _mm_body,
        grid=(NP // BM,),
        in_specs=[
            pl.BlockSpec((BM, 128), lambda i: (i, 0)),
            pl.BlockSpec((128, 128), lambda i: (0, 0)),
        ],
        out_specs=pl.BlockSpec((BM, 128), lambda i: (i, 0)),
        out_shape=jax.ShapeDtypeStruct((NP, 128), jnp.float32),
    )(x_pad, W1)


def _enc_a_body(h1_ref, degp_ref, g1_ref, dinv_ref):
    deg = degp_ref[0, :, :16] + degp_ref[1, :, :16] + 1.0
    dinv = lax.rsqrt(deg)
    dinv_ref[...] = dinv
    g1_ref[...] = h1_ref[...] * dinv[:, 0:1]


def _enc_a_call(h1, degp):
    return pl.pallas_call(
        _enc_a_body,
        grid=(NP // BM,),
        in_specs=[
            pl.BlockSpec((BM, 128), lambda i: (i, 0)),
            pl.BlockSpec((2, BM, 128), lambda i: (0, i, 0)),
        ],
        out_specs=[
            pl.BlockSpec((BM, 128), lambda i: (i, 0)),
            pl.BlockSpec((BM, 16), lambda i: (i, 0)),
        ],
        out_shape=[
            jax.ShapeDtypeStruct((NP, 128), jnp.float32),
            jax.ShapeDtypeStruct((NP, 16), jnp.float32),
        ],
    )(h1, degp)


def _enc_b_body(s1p_ref, g1_ref, dinv_ref, b1_ref, wc_ref, g2_ref):
    dinv = dinv_ref[...][:, 0:1]
    h = s1p_ref[0] + s1p_ref[1] + g1_ref[...]
    h = jnp.maximum(h * dinv + b1_ref[...], 0.0)
    c2 = jnp.dot(h, wc_ref[...], preferred_element_type=jnp.float32)
    g2_ref[...] = c2 * dinv


def _enc_b_call(s1p, g1, dinv16, b1r, Wc):
    return pl.pallas_call(
        _enc_b_body,
        grid=(NP // BM,),
        in_specs=[
            pl.BlockSpec((2, BM, 128), lambda i: (0, i, 0)),
            pl.BlockSpec((BM, 128), lambda i: (i, 0)),
            pl.BlockSpec((BM, 16), lambda i: (i, 0)),
            pl.BlockSpec((1, 128), lambda i: (0, 0)),
            pl.BlockSpec((128, 128), lambda i: (0, 0)),
        ],
        out_specs=pl.BlockSpec((BM, 128), lambda i: (i, 0)),
        out_shape=jax.ShapeDtypeStruct((NP, 128), jnp.float32),
    )(s1p, g1, dinv16, b1r, Wc)


def _enc_c_body(s2p_ref, g2_ref, dinv_ref, bc_ref, out_ref):
    dinv = dinv_ref[...][:, 0:1]
    t = s2p_ref[0] + s2p_ref[1] + g2_ref[...]
    out_ref[...] = t * dinv + bc_ref[...]


def _enc_c_call(s2p, g2, dinv16, bcr):
    return pl.pallas_call(
        _enc_c_body,
        grid=(NP // BM,),
        in_specs=[
            pl.BlockSpec((2, BM, 128), lambda i: (0, i, 0)),
            pl.BlockSpec((BM, 128), lambda i: (i, 0)),
            pl.BlockSpec((BM, 16), lambda i: (i, 0)),
            pl.BlockSpec((1, 128), lambda i: (0, 0)),
        ],
        out_specs=pl.BlockSpec((BM, 128), lambda i: (i, 0)),
        out_shape=jax.ShapeDtypeStruct((NP, 128), jnp.float32),
    )(s2p, g2, dinv16, bcr)


DM = 512
DN = 1024


def _dec_body(a_ref, b_ref, o_ref):
    acc = lax.dot_general(a_ref[...], b_ref[...],
                          (((1,), (1,)), ((), ())),
                          preferred_element_type=jnp.float32)
    o_ref[...] = jax.nn.sigmoid(acc)


def _dec_call(z64):
    return pl.pallas_call(
        _dec_body,
        grid=(NP // DM, NP // DN),
        in_specs=[
            pl.BlockSpec((DM, 64), lambda i, j: (i, 0)),
            pl.BlockSpec((DN, 64), lambda i, j: (j, 0)),
        ],
        out_specs=pl.BlockSpec((DM, DN), lambda i, j: (i, j)),
        out_shape=jax.ShapeDtypeStruct((N, N), jnp.float32),
    )(z64, z64)


def kernel(x, edge_index, W1, b1, Wmu, bmu, Wls, bls):
    src = edge_index[0].astype(jnp.int32)
    dst = edge_index[1].astype(jnp.int32)
    pad = jnp.full((EP - E,), N, jnp.int32)
    src_f = jnp.concatenate([src, pad]).reshape(NCHUNK, CH)
    dst_f = jnp.concatenate([dst, pad]).reshape(NCHUNK, CH)
    dst_r = dst_f.reshape(NW, SLABS, CH)
    x_pad = jnp.pad(x, ((0, NP - N), (0, 0)))
    Wc = jnp.concatenate([Wmu, Wls], axis=1)
    bc = jnp.concatenate([bmu, bls]).reshape(1, 128)
    b1r = b1.reshape(1, 128)
    ones128 = jnp.ones((CH, 128), jnp.float32)
    zeros128 = jnp.zeros((ROWS_PER_TILE, 128), jnp.float32)
    zeros632 = jnp.zeros((ACC_RPT, 128), jnp.float32)

    degp = _deg_call(dst_r, ones128, zeros128)
    h1 = _mm_call(x_pad, W1)
    g1, dinv16 = _enc_a_call(h1, degp)
    s1p = _prop_call(g1, src_f, dst_f, zeros632)
    g2 = _enc_b_call(s1p, g1, dinv16, b1r, Wc)
    s2p = _prop_call(g2, src_f, dst_f, zeros632)
    full2 = _enc_c_call(s2p, g2, dinv16, bc)

    mu = full2[:N, :64]
    logstd = full2[:N, 64:]
    adj = _dec_call(full2[:, :64])
    return adj, mu, logstd


# speedup vs baseline: 6.9751x; 1.0365x over previous
import functools

import jax
import jax.numpy as jnp
from jax import lax
from jax.experimental import pallas as pl
from jax.experimental.pallas import tpu as pltpu
from jax.experimental.pallas import tpu_sc as plsc

N = 10000
NP = 10240
E = 160000
NW = 32
CH = 128
SLABS = 40
EP = NW * SLABS * CH
ROWS_PER_TILE = NP // 16
BM = 512


def _mesh():
    return plsc.VectorSubcoreMesh(core_axis_name="c", subcore_axis_name="s")


def _deg_body(dst_hbm, ones_hbm, zeros_hbm, out_hbm, idx_v, ones_v, dsem,
              acc_sh):
    c = lax.axis_index("c")
    s = lax.axis_index("s")
    w = c * 16 + s
    base = s * ROWS_PER_TILE
    pltpu.sync_copy(zeros_hbm, acc_sh.at[pl.ds(base, ROWS_PER_TILE)])
    pltpu.sync_copy(dst_hbm.at[w], idx_v)
    pltpu.sync_copy(ones_hbm, ones_v)
    plsc.subcore_barrier()

    def body(g, carry):
        descs = [
            pltpu.async_copy(ones_v, acc_sh.at[idx_v.at[g * 8 + b]],
                             dsem, add=True)
            for b in range(8)
        ]
        for d in descs:
            d.wait()
        return carry

    lax.fori_loop(0, SLABS // 8, body, 0)
    plsc.subcore_barrier()
    pltpu.sync_copy(acc_sh.at[pl.ds(base, ROWS_PER_TILE)],
                    out_hbm.at[c, pl.ds(base, ROWS_PER_TILE)])


def _deg_call(dst_r, ones128, zeros128):
    k = functools.partial(
        pl.kernel,
        mesh=_mesh(),
        out_type=jax.ShapeDtypeStruct((2, NP, 128), jnp.float32),
        scratch_types=[
            pltpu.VMEM((SLABS, CH), jnp.int32),
            pltpu.VMEM((CH, 128), jnp.float32),
            pltpu.SemaphoreType.DMA,
            pltpu.VMEM_SHARED((NP, 128), jnp.float32),
        ],
    )(_deg_body)
    return k(dst_r, ones128, zeros128)


NBUF = 2
NCHUNK = EP // CH
SA = 64
SB = 80 - SA
FAST_C = 1
ACC_RPT = 632
ACC_ROWS = 16 * ACC_RPT


def _prop_body(g_hbm, src_hbm, dst_hbm, zeros_hbm, out_hbm,
               sidx_v, didx_v, rows_v, gsem, ssem, acc_sh):
    c = lax.axis_index("c")
    s = lax.axis_index("s")
    base = s * ACC_RPT
    pltpu.sync_copy(zeros_hbm, acc_sh.at[pl.ds(base, ACC_RPT)])

    def run(start, n_chunks):
        pltpu.sync_copy(src_hbm.at[pl.ds(start, n_chunks)],
                        sidx_v.at[pl.ds(0, n_chunks)])
        pltpu.sync_copy(dst_hbm.at[pl.ds(start, n_chunks)],
                        didx_v.at[pl.ds(0, n_chunks)])
        plsc.subcore_barrier()

        def body(g, carry):
            gd = [
                pltpu.async_copy(g_hbm.at[sidx_v.at[g * NBUF + b]],
                                 rows_v.at[b], gsem.at[b])
                for b in range(NBUF)
            ]
            sd = []
            for b in range(NBUF):
                gd[b].wait()
                sd.append(pltpu.async_copy(
                    rows_v.at[b], acc_sh.at[didx_v.at[g * NBUF + b]],
                    ssem.at[b], add=True))
            for d in sd:
                d.wait()
            return carry

        lax.fori_loop(0, n_chunks // NBUF, body, 0)

    @pl.when(c == FAST_C)
    def _():
        run(s * SA, SA)

    @pl.when(c != FAST_C)
    def _():
        run(16 * SA + s * SB, SB)

    plsc.subcore_barrier()
    pltpu.sync_copy(acc_sh.at[pl.ds(base, ACC_RPT)],
                    out_hbm.at[c, pl.ds(base, ACC_RPT)])


def _prop_call(g, src_f, dst_f, zeros128):
    k = functools.partial(
        pl.kernel,
        mesh=_mesh(),
        out_type=jax.ShapeDtypeStruct((2, ACC_ROWS, 128), jnp.float32),
        scratch_types=[
            pltpu.VMEM((SA, CH), jnp.int32),
            pltpu.VMEM((SA, CH), jnp.int32),
            pltpu.VMEM((NBUF, CH, 128), jnp.float32),
            pltpu.SemaphoreType.DMA((NBUF,)),
            pltpu.SemaphoreType.DMA((NBUF,)),
            pltpu.VMEM_SHARED((ACC_ROWS, 128), jnp.float32),
        ],
    )(_prop_body)
    return k(g, src_f, dst_f, zeros128)


def _mm_body(x_ref, w_ref, o_ref):
    o_ref[...] = jnp.dot(x_ref[...], w_ref[...],
                         preferred_element_type=jnp.float32)


def _mm_call(x_pad, W1):
    return pl.pallas_call(
        _mm_body,
        grid=(NP // BM,),
        in_specs=[
            pl.BlockSpec((BM, 128), lambda i: (i, 0)),
            pl.BlockSpec((128, 128), lambda i: (0, 0)),
        ],
        out_specs=pl.BlockSpec((BM, 128), lambda i: (i, 0)),
        out_shape=jax.ShapeDtypeStruct((NP, 128), jnp.float32),
    )(x_pad, W1)


def _enc_a_body(h1_ref, degp_ref, g1_ref, dinv_ref):
    deg = degp_ref[0, :, :16] + degp_ref[1, :, :16] + 1.0
    dinv = lax.rsqrt(deg)
    dinv_ref[...] = dinv
    g1_ref[...] = h1_ref[...] * dinv[:, 0:1]


def _enc_a_call(h1, degp):
    return pl.pallas_call(
        _enc_a_body,
        grid=(NP // BM,),
        in_specs=[
            pl.BlockSpec((BM, 128), lambda i: (i, 0)),
            pl.BlockSpec((2, BM, 128), lambda i: (0, i, 0)),
        ],
        out_specs=[
            pl.BlockSpec((BM, 128), lambda i: (i, 0)),
            pl.BlockSpec((BM, 16), lambda i: (i, 0)),
        ],
        out_shape=[
            jax.ShapeDtypeStruct((NP, 128), jnp.float32),
            jax.ShapeDtypeStruct((NP, 16), jnp.float32),
        ],
    )(h1, degp)


def _enc_b_body(s1p_ref, g1_ref, dinv_ref, b1_ref, wc_ref, g2_ref):
    dinv = dinv_ref[...][:, 0:1]
    h = s1p_ref[0] + s1p_ref[1] + g1_ref[...]
    h = jnp.maximum(h * dinv + b1_ref[...], 0.0)
    c2 = jnp.dot(h, wc_ref[...], preferred_element_type=jnp.float32)
    g2_ref[...] = c2 * dinv


def _enc_b_call(s1p, g1, dinv16, b1r, Wc):
    return pl.pallas_call(
        _enc_b_body,
        grid=(NP // BM,),
        in_specs=[
            pl.BlockSpec((2, BM, 128), lambda i: (0, i, 0)),
            pl.BlockSpec((BM, 128), lambda i: (i, 0)),
            pl.BlockSpec((BM, 16), lambda i: (i, 0)),
            pl.BlockSpec((1, 128), lambda i: (0, 0)),
            pl.BlockSpec((128, 128), lambda i: (0, 0)),
        ],
        out_specs=pl.BlockSpec((BM, 128), lambda i: (i, 0)),
        out_shape=jax.ShapeDtypeStruct((NP, 128), jnp.float32),
    )(s1p, g1, dinv16, b1r, Wc)


def _enc_c_body(s2p_ref, g2_ref, dinv_ref, bc_ref, out_ref):
    dinv = dinv_ref[...][:, 0:1]
    t = s2p_ref[0] + s2p_ref[1] + g2_ref[...]
    out_ref[...] = t * dinv + bc_ref[...]


def _enc_c_call(s2p, g2, dinv16, bcr):
    return pl.pallas_call(
        _enc_c_body,
        grid=(NP // BM,),
        in_specs=[
            pl.BlockSpec((2, BM, 128), lambda i: (0, i, 0)),
            pl.BlockSpec((BM, 128), lambda i: (i, 0)),
            pl.BlockSpec((BM, 16), lambda i: (i, 0)),
            pl.BlockSpec((1, 128), lambda i: (0, 0)),
        ],
        out_specs=pl.BlockSpec((BM, 128), lambda i: (i, 0)),
        out_shape=jax.ShapeDtypeStruct((NP, 128), jnp.float32),
    )(s2p, g2, dinv16, bcr)


DM = 512
DN = 1024


def _dec_body(a_ref, b_ref, o_ref):
    acc = lax.dot_general(a_ref[...], b_ref[...],
                          (((1,), (1,)), ((), ())),
                          preferred_element_type=jnp.float32)
    o_ref[...] = jax.nn.sigmoid(acc)


def _dec_call(z64):
    return pl.pallas_call(
        _dec_body,
        grid=(NP // DM, NP // DN),
        in_specs=[
            pl.BlockSpec((DM, 64), lambda i, j: (i, 0)),
            pl.BlockSpec((DN, 64), lambda i, j: (j, 0)),
        ],
        out_specs=pl.BlockSpec((DM, DN), lambda i, j: (i, j)),
        out_shape=jax.ShapeDtypeStruct((N, N), jnp.float32),
    )(z64, z64)


def kernel(x, edge_index, W1, b1, Wmu, bmu, Wls, bls):
    src = edge_index[0].astype(jnp.int32)
    dst = edge_index[1].astype(jnp.int32)
    pad = jnp.full((EP - E,), N, jnp.int32)
    src_f = jnp.concatenate([src, pad]).reshape(NCHUNK, CH)
    dst_f = jnp.concatenate([dst, pad]).reshape(NCHUNK, CH)
    dst_r = dst_f.reshape(NW, SLABS, CH)
    x_pad = jnp.pad(x, ((0, NP - N), (0, 0)))
    Wc = jnp.concatenate([Wmu, Wls], axis=1)
    bc = jnp.concatenate([bmu, bls]).reshape(1, 128)
    b1r = b1.reshape(1, 128)
    ones128 = jnp.ones((CH, 128), jnp.float32)
    zeros128 = jnp.zeros((ROWS_PER_TILE, 128), jnp.float32)
    zeros632 = jnp.zeros((ACC_RPT, 128), jnp.float32)

    degp = _deg_call(dst_r, ones128, zeros128)
    h1 = _mm_call(x_pad, W1)
    g1, dinv16 = _enc_a_call(h1, degp)
    s1p = _prop_call(g1, src_f, dst_f, zeros632)
    g2 = _enc_b_call(s1p, g1, dinv16, b1r, Wc)
    s2p = _prop_call(g2, src_f, dst_f, zeros632)
    full2 = _enc_c_call(s2p, g2, dinv16, bc)

    mu = full2[:N, :64]
    logstd = full2[:N, 64:]
    adj = _dec_call(full2[:, :64])
    return adj, mu, logstd


# speedup vs baseline: 7.4327x; 1.0656x over previous
import functools

import jax
import jax.numpy as jnp
from jax import lax
from jax.experimental import pallas as pl
from jax.experimental.pallas import tpu as pltpu
from jax.experimental.pallas import tpu_sc as plsc

N = 10000
NP = 10240
E = 160000
NW = 32
CH = 128
SLABS = 40
EP = NW * SLABS * CH
ROWS_PER_TILE = NP // 16
BM = 512


def _mesh():
    return plsc.VectorSubcoreMesh(core_axis_name="c", subcore_axis_name="s")


def _deg_body(dst_hbm, ones_hbm, zeros_hbm, out_hbm, idx_v, ones_v, dsem,
              acc_sh):
    c = lax.axis_index("c")
    s = lax.axis_index("s")
    w = c * 16 + s
    base = s * ROWS_PER_TILE
    pltpu.sync_copy(zeros_hbm, acc_sh.at[pl.ds(base, ROWS_PER_TILE)])
    pltpu.sync_copy(dst_hbm.at[w], idx_v)
    pltpu.sync_copy(ones_hbm, ones_v)
    plsc.subcore_barrier()

    def body(g, carry):
        descs = [
            pltpu.async_copy(ones_v, acc_sh.at[idx_v.at[g * 8 + b]],
                             dsem, add=True)
            for b in range(8)
        ]
        for d in descs:
            d.wait()
        return carry

    lax.fori_loop(0, SLABS // 8, body, 0)
    plsc.subcore_barrier()
    pltpu.sync_copy(acc_sh.at[pl.ds(base, ROWS_PER_TILE)],
                    out_hbm.at[c, pl.ds(base, ROWS_PER_TILE)])


def _deg_call(dst_r, ones128, zeros128):
    k = functools.partial(
        pl.kernel,
        mesh=_mesh(),
        out_type=jax.ShapeDtypeStruct((2, NP, 128), jnp.float32),
        scratch_types=[
            pltpu.VMEM((SLABS, CH), jnp.int32),
            pltpu.VMEM((CH, 128), jnp.float32),
            pltpu.SemaphoreType.DMA,
            pltpu.VMEM_SHARED((NP, 128), jnp.float32),
        ],
    )(_deg_body)
    return k(dst_r, ones128, zeros128)


NBUF = 2
NCHUNK = EP // CH
SA = 64
SB = 80 - SA
FAST_C = 1
ACC_RPT = 632
ACC_ROWS = 16 * ACC_RPT


def _prop_body(g_hbm, src_hbm, dst_hbm, zeros_hbm, out_hbm,
               sidx_v, didx_v, rows_v, gsem, ssem, acc_sh):
    c = lax.axis_index("c")
    s = lax.axis_index("s")
    base = s * ACC_RPT
    pltpu.sync_copy(zeros_hbm, acc_sh.at[pl.ds(base, ACC_RPT)])

    def run(start, n_chunks):
        pltpu.sync_copy(src_hbm.at[pl.ds(start, n_chunks)],
                        sidx_v.at[pl.ds(0, n_chunks)])
        pltpu.sync_copy(dst_hbm.at[pl.ds(start, n_chunks)],
                        didx_v.at[pl.ds(0, n_chunks)])
        plsc.subcore_barrier()

        def body(g, carry):
            gd = [
                pltpu.async_copy(g_hbm.at[sidx_v.at[g * NBUF + b]],
                                 rows_v.at[b], gsem.at[b])
                for b in range(NBUF)
            ]
            sd = []
            for b in range(NBUF):
                gd[b].wait()
                sd.append(pltpu.async_copy(
                    rows_v.at[b], acc_sh.at[didx_v.at[g * NBUF + b]],
                    ssem.at[b], add=True))
            for d in sd:
                d.wait()
            return carry

        lax.fori_loop(0, n_chunks // NBUF, body, 0)

    @pl.when(c == FAST_C)
    def _():
        run(s * SA, SA)

    @pl.when(c != FAST_C)
    def _():
        run(16 * SA + s * SB, SB)

    plsc.subcore_barrier()
    pltpu.sync_copy(acc_sh.at[pl.ds(base, ACC_RPT)],
                    out_hbm.at[c, pl.ds(base, ACC_RPT)])


def _prop_call(g, src_f, dst_f, zeros128):
    k = functools.partial(
        pl.kernel,
        mesh=_mesh(),
        out_type=jax.ShapeDtypeStruct((2, ACC_ROWS, 128), jnp.float32),
        scratch_types=[
            pltpu.VMEM((SA, CH), jnp.int32),
            pltpu.VMEM((SA, CH), jnp.int32),
            pltpu.VMEM((NBUF, CH, 128), jnp.float32),
            pltpu.SemaphoreType.DMA((NBUF,)),
            pltpu.SemaphoreType.DMA((NBUF,)),
            pltpu.VMEM_SHARED((ACC_ROWS, 128), jnp.float32),
        ],
    )(_prop_body)
    return k(g, src_f, dst_f, zeros128)


def _mm_body(x_ref, w_ref, o_ref):
    o_ref[...] = jnp.dot(x_ref[...], w_ref[...],
                         preferred_element_type=jnp.float32)


def _mm_call(x_pad, W1):
    return pl.pallas_call(
        _mm_body,
        grid=(NP // BM,),
        in_specs=[
            pl.BlockSpec((BM, 128), lambda i: (i, 0)),
            pl.BlockSpec((128, 128), lambda i: (0, 0)),
        ],
        out_specs=pl.BlockSpec((BM, 128), lambda i: (i, 0)),
        out_shape=jax.ShapeDtypeStruct((NP, 128), jnp.float32),
    )(x_pad, W1)


def _enc_a_body(h1_ref, degp_ref, g1_ref, dinv_ref):
    deg = degp_ref[0, :, :16] + degp_ref[1, :, :16] + 1.0
    dinv = lax.rsqrt(deg)
    dinv_ref[...] = dinv
    g1_ref[...] = h1_ref[...] * dinv[:, 0:1]


def _enc_a_call(h1, degp):
    return pl.pallas_call(
        _enc_a_body,
        grid=(NP // BM,),
        in_specs=[
            pl.BlockSpec((BM, 128), lambda i: (i, 0)),
            pl.BlockSpec((2, BM, 128), lambda i: (0, i, 0)),
        ],
        out_specs=[
            pl.BlockSpec((BM, 128), lambda i: (i, 0)),
            pl.BlockSpec((BM, 16), lambda i: (i, 0)),
        ],
        out_shape=[
            jax.ShapeDtypeStruct((NP, 128), jnp.float32),
            jax.ShapeDtypeStruct((NP, 16), jnp.float32),
        ],
    )(h1, degp)


def _enc_b_body(s1p_ref, g1_ref, dinv_ref, b1_ref, wc_ref, g2_ref):
    dinv = dinv_ref[...][:, 0:1]
    h = s1p_ref[0] + s1p_ref[1] + g1_ref[...]
    h = jnp.maximum(h * dinv + b1_ref[...], 0.0)
    c2 = jnp.dot(h, wc_ref[...], preferred_element_type=jnp.float32)
    g2_ref[...] = c2 * dinv


def _enc_b_call(s1p, g1, dinv16, b1r, Wc):
    return pl.pallas_call(
        _enc_b_body,
        grid=(NP // BM,),
        in_specs=[
            pl.BlockSpec((2, BM, 128), lambda i: (0, i, 0)),
            pl.BlockSpec((BM, 128), lambda i: (i, 0)),
            pl.BlockSpec((BM, 16), lambda i: (i, 0)),
            pl.BlockSpec((1, 128), lambda i: (0, 0)),
            pl.BlockSpec((128, 128), lambda i: (0, 0)),
        ],
        out_specs=pl.BlockSpec((BM, 128), lambda i: (i, 0)),
        out_shape=jax.ShapeDtypeStruct((NP, 128), jnp.float32),
    )(s1p, g1, dinv16, b1r, Wc)


def _enc_c_body(s2p_ref, g2_ref, dinv_ref, bc_ref, out_ref):
    dinv = dinv_ref[...][:, 0:1]
    t = s2p_ref[0] + s2p_ref[1] + g2_ref[...]
    out_ref[...] = t * dinv + bc_ref[...]


def _enc_c_call(s2p, g2, dinv16, bcr):
    return pl.pallas_call(
        _enc_c_body,
        grid=(NP // BM,),
        in_specs=[
            pl.BlockSpec((2, BM, 128), lambda i: (0, i, 0)),
            pl.BlockSpec((BM, 128), lambda i: (i, 0)),
            pl.BlockSpec((BM, 16), lambda i: (i, 0)),
            pl.BlockSpec((1, 128), lambda i: (0, 0)),
        ],
        out_specs=pl.BlockSpec((BM, 128), lambda i: (i, 0)),
        out_shape=jax.ShapeDtypeStruct((NP, 128), jnp.float32),
    )(s2p, g2, dinv16, bcr)


DM = 512
DN = 2048


def _dec_body(a_ref, b_ref, o_ref):
    acc = lax.dot_general(a_ref[...], b_ref[...],
                          (((1,), (1,)), ((), ())),
                          preferred_element_type=jnp.float32)
    o_ref[...] = jax.nn.sigmoid(acc)


def _dec_call(z64):
    return pl.pallas_call(
        _dec_body,
        grid=(NP // DM, NP // DN),
        in_specs=[
            pl.BlockSpec((DM, 64), lambda i, j: (i, 0)),
            pl.BlockSpec((DN, 64), lambda i, j: (j, 0)),
        ],
        out_specs=pl.BlockSpec((DM, DN), lambda i, j: (i, j)),
        out_shape=jax.ShapeDtypeStruct((N, N), jnp.float32),
    )(z64, z64)


def kernel(x, edge_index, W1, b1, Wmu, bmu, Wls, bls):
    src = edge_index[0].astype(jnp.int32)
    dst = edge_index[1].astype(jnp.int32)
    pad = jnp.full((EP - E,), N, jnp.int32)
    src_f = jnp.concatenate([src, pad]).reshape(NCHUNK, CH)
    dst_f = jnp.concatenate([dst, pad]).reshape(NCHUNK, CH)
    dst_r = dst_f.reshape(NW, SLABS, CH)
    x_pad = jnp.pad(x, ((0, NP - N), (0, 0)))
    Wc = jnp.concatenate([Wmu, Wls], axis=1)
    bc = jnp.concatenate([bmu, bls]).reshape(1, 128)
    b1r = b1.reshape(1, 128)
    ones128 = jnp.ones((CH, 128), jnp.float32)
    zeros128 = jnp.zeros((ROWS_PER_TILE, 128), jnp.float32)
    zeros632 = jnp.zeros((ACC_RPT, 128), jnp.float32)

    degp = _deg_call(dst_r, ones128, zeros128)
    h1 = _mm_call(x_pad, W1)
    g1, dinv16 = _enc_a_call(h1, degp)
    s1p = _prop_call(g1, src_f, dst_f, zeros632)
    g2 = _enc_b_call(s1p, g1, dinv16, b1r, Wc)
    s2p = _prop_call(g2, src_f, dst_f, zeros632)
    full2 = _enc_c_call(s2p, g2, dinv16, bc)

    mu = full2[:N, :64]
    logstd = full2[:N, 64:]
    adj = _dec_call(full2[:, :64])
    return adj, mu, logstd


# speedup vs baseline: 7.8585x; 1.0573x over previous
import functools

import jax
import jax.numpy as jnp
from jax import lax
from jax.experimental import pallas as pl
from jax.experimental.pallas import tpu as pltpu
from jax.experimental.pallas import tpu_sc as plsc

N = 10000
NP = 10240
E = 160000
NW = 32
CH = 128
SLABS = 40
EP = NW * SLABS * CH
ROWS_PER_TILE = NP // 16
BM = 512


def _mesh():
    return plsc.VectorSubcoreMesh(core_axis_name="c", subcore_axis_name="s")


def _deg_body(dst_hbm, ones_hbm, zeros_hbm, out_hbm, idx_v, ones_v, dsem,
              acc_sh):
    c = lax.axis_index("c")
    s = lax.axis_index("s")
    w = c * 16 + s
    base = s * ROWS_PER_TILE
    pltpu.sync_copy(zeros_hbm, acc_sh.at[pl.ds(base, ROWS_PER_TILE)])
    pltpu.sync_copy(dst_hbm.at[w], idx_v)
    pltpu.sync_copy(ones_hbm, ones_v)
    plsc.subcore_barrier()

    def body(g, carry):
        descs = [
            pltpu.async_copy(ones_v, acc_sh.at[idx_v.at[g * 8 + b]],
                             dsem, add=True)
            for b in range(8)
        ]
        for d in descs:
            d.wait()
        return carry

    lax.fori_loop(0, SLABS // 8, body, 0)
    plsc.subcore_barrier()
    pltpu.sync_copy(acc_sh.at[pl.ds(base, ROWS_PER_TILE)],
                    out_hbm.at[c, pl.ds(base, ROWS_PER_TILE)])


def _deg_call(dst_r, ones128, zeros128):
    k = functools.partial(
        pl.kernel,
        mesh=_mesh(),
        out_type=jax.ShapeDtypeStruct((2, NP, 128), jnp.float32),
        scratch_types=[
            pltpu.VMEM((SLABS, CH), jnp.int32),
            pltpu.VMEM((CH, 128), jnp.float32),
            pltpu.SemaphoreType.DMA,
            pltpu.VMEM_SHARED((NP, 128), jnp.float32),
        ],
    )(_deg_body)
    return k(dst_r, ones128, zeros128)


NBUF = 2
NCHUNK = EP // CH
SA = 64
SB = 80 - SA
FAST_C = 1
ACC_RPT = 632
ACC_ROWS = 16 * ACC_RPT


def _prop_body(g_hbm, src_hbm, dst_hbm, zeros_hbm, out_hbm,
               sidx_v, didx_v, rows_v, gsem, ssem, acc_sh):
    c = lax.axis_index("c")
    s = lax.axis_index("s")
    base = s * ACC_RPT
    pltpu.sync_copy(zeros_hbm, acc_sh.at[pl.ds(base, ACC_RPT)])

    def run(start, n_chunks):
        pltpu.sync_copy(src_hbm.at[pl.ds(start, n_chunks)],
                        sidx_v.at[pl.ds(0, n_chunks)])
        pltpu.sync_copy(dst_hbm.at[pl.ds(start, n_chunks)],
                        didx_v.at[pl.ds(0, n_chunks)])
        plsc.subcore_barrier()

        def body(g, carry):
            gd = [
                pltpu.async_copy(g_hbm.at[sidx_v.at[g * NBUF + b]],
                                 rows_v.at[b], gsem.at[b])
                for b in range(NBUF)
            ]
            sd = []
            for b in range(NBUF):
                gd[b].wait()
                sd.append(pltpu.async_copy(
                    rows_v.at[b], acc_sh.at[didx_v.at[g * NBUF + b]],
                    ssem.at[b], add=True))
            for d in sd:
                d.wait()
            return carry

        lax.fori_loop(0, n_chunks // NBUF, body, 0)

    @pl.when(c == FAST_C)
    def _():
        run(s * SA, SA)

    @pl.when(c != FAST_C)
    def _():
        run(16 * SA + s * SB, SB)

    plsc.subcore_barrier()
    pltpu.sync_copy(acc_sh.at[pl.ds(base, ACC_RPT)],
                    out_hbm.at[c, pl.ds(base, ACC_RPT)])


def _prop_call(g, src_f, dst_f, zeros128):
    k = functools.partial(
        pl.kernel,
        mesh=_mesh(),
        out_type=jax.ShapeDtypeStruct((2, ACC_ROWS, 128), jnp.float32),
        scratch_types=[
            pltpu.VMEM((SA, CH), jnp.int32),
            pltpu.VMEM((SA, CH), jnp.int32),
            pltpu.VMEM((NBUF, CH, 128), jnp.float32),
            pltpu.SemaphoreType.DMA((NBUF,)),
            pltpu.SemaphoreType.DMA((NBUF,)),
            pltpu.VMEM_SHARED((ACC_ROWS, 128), jnp.float32),
        ],
    )(_prop_body)
    return k(g, src_f, dst_f, zeros128)


def _mm_body(x_ref, w_ref, o_ref):
    o_ref[...] = jnp.dot(x_ref[...], w_ref[...],
                         preferred_element_type=jnp.float32)


def _mm_call(x_pad, W1):
    return pl.pallas_call(
        _mm_body,
        grid=(NP // BM,),
        in_specs=[
            pl.BlockSpec((BM, 128), lambda i: (i, 0)),
            pl.BlockSpec((128, 128), lambda i: (0, 0)),
        ],
        out_specs=pl.BlockSpec((BM, 128), lambda i: (i, 0)),
        out_shape=jax.ShapeDtypeStruct((NP, 128), jnp.float32),
    )(x_pad, W1)


def _enc_a_body(h1_ref, degp_ref, g1_ref, dinv_ref):
    deg = degp_ref[0, :, :16] + degp_ref[1, :, :16] + 1.0
    dinv = lax.rsqrt(deg)
    dinv_ref[...] = dinv
    g1_ref[...] = h1_ref[...] * dinv[:, 0:1]


def _enc_a_call(h1, degp):
    return pl.pallas_call(
        _enc_a_body,
        grid=(NP // BM,),
        in_specs=[
            pl.BlockSpec((BM, 128), lambda i: (i, 0)),
            pl.BlockSpec((2, BM, 128), lambda i: (0, i, 0)),
        ],
        out_specs=[
            pl.BlockSpec((BM, 128), lambda i: (i, 0)),
            pl.BlockSpec((BM, 16), lambda i: (i, 0)),
        ],
        out_shape=[
            jax.ShapeDtypeStruct((NP, 128), jnp.float32),
            jax.ShapeDtypeStruct((NP, 16), jnp.float32),
        ],
    )(h1, degp)


def _enc_b_body(s1p_ref, g1_ref, dinv_ref, b1_ref, wc_ref, g2_ref):
    dinv = dinv_ref[...][:, 0:1]
    h = s1p_ref[0] + s1p_ref[1] + g1_ref[...]
    h = jnp.maximum(h * dinv + b1_ref[...], 0.0)
    c2 = jnp.dot(h, wc_ref[...], preferred_element_type=jnp.float32)
    g2_ref[...] = c2 * dinv


def _enc_b_call(s1p, g1, dinv16, b1r, Wc):
    return pl.pallas_call(
        _enc_b_body,
        grid=(NP // BM,),
        in_specs=[
            pl.BlockSpec((2, BM, 128), lambda i: (0, i, 0)),
            pl.BlockSpec((BM, 128), lambda i: (i, 0)),
            pl.BlockSpec((BM, 16), lambda i: (i, 0)),
            pl.BlockSpec((1, 128), lambda i: (0, 0)),
            pl.BlockSpec((128, 128), lambda i: (0, 0)),
        ],
        out_specs=pl.BlockSpec((BM, 128), lambda i: (i, 0)),
        out_shape=jax.ShapeDtypeStruct((NP, 128), jnp.float32),
    )(s1p, g1, dinv16, b1r, Wc)


def _enc_c_body(s2p_ref, g2_ref, dinv_ref, bc_ref, out_ref):
    dinv = dinv_ref[...][:, 0:1]
    t = s2p_ref[0] + s2p_ref[1] + g2_ref[...]
    out_ref[...] = t * dinv + bc_ref[...]


def _enc_c_call(s2p, g2, dinv16, bcr):
    return pl.pallas_call(
        _enc_c_body,
        grid=(NP // BM,),
        in_specs=[
            pl.BlockSpec((2, BM, 128), lambda i: (0, i, 0)),
            pl.BlockSpec((BM, 128), lambda i: (i, 0)),
            pl.BlockSpec((BM, 16), lambda i: (i, 0)),
            pl.BlockSpec((1, 128), lambda i: (0, 0)),
        ],
        out_specs=pl.BlockSpec((BM, 128), lambda i: (i, 0)),
        out_shape=jax.ShapeDtypeStruct((NP, 128), jnp.float32),
    )(s2p, g2, dinv16, bcr)


DM = 1024
DN = 2048


def _dec_body(a_ref, b_ref, o_ref):
    acc = lax.dot_general(a_ref[...], b_ref[...],
                          (((1,), (1,)), ((), ())),
                          preferred_element_type=jnp.float32)
    o_ref[...] = jax.nn.sigmoid(acc)


def _dec_call(z64):
    return pl.pallas_call(
        _dec_body,
        grid=(NP // DM, NP // DN),
        in_specs=[
            pl.BlockSpec((DM, 64), lambda i, j: (i, 0)),
            pl.BlockSpec((DN, 64), lambda i, j: (j, 0)),
        ],
        out_specs=pl.BlockSpec((DM, DN), lambda i, j: (i, j)),
        out_shape=jax.ShapeDtypeStruct((N, N), jnp.float32),
    )(z64, z64)


def kernel(x, edge_index, W1, b1, Wmu, bmu, Wls, bls):
    src = edge_index[0].astype(jnp.int32)
    dst = edge_index[1].astype(jnp.int32)
    pad = jnp.full((EP - E,), N, jnp.int32)
    src_f = jnp.concatenate([src, pad]).reshape(NCHUNK, CH)
    dst_f = jnp.concatenate([dst, pad]).reshape(NCHUNK, CH)
    dst_r = dst_f.reshape(NW, SLABS, CH)
    x_pad = jnp.pad(x, ((0, NP - N), (0, 0)))
    Wc = jnp.concatenate([Wmu, Wls], axis=1)
    bc = jnp.concatenate([bmu, bls]).reshape(1, 128)
    b1r = b1.reshape(1, 128)
    ones128 = jnp.ones((CH, 128), jnp.float32)
    zeros128 = jnp.zeros((ROWS_PER_TILE, 128), jnp.float32)
    zeros632 = jnp.zeros((ACC_RPT, 128), jnp.float32)

    degp = _deg_call(dst_r, ones128, zeros128)
    h1 = _mm_call(x_pad, W1)
    g1, dinv16 = _enc_a_call(h1, degp)
    s1p = _prop_call(g1, src_f, dst_f, zeros632)
    g2 = _enc_b_call(s1p, g1, dinv16, b1r, Wc)
    s2p = _prop_call(g2, src_f, dst_f, zeros632)
    full2 = _enc_c_call(s2p, g2, dinv16, bc)

    mu = full2[:N, :64]
    logstd = full2[:N, 64:]
    adj = _dec_call(full2[:, :64])
    return adj, mu, logstd


# speedup vs baseline: 8.0151x; 1.0199x over previous
import functools

import jax
import jax.numpy as jnp
from jax import lax
from jax.experimental import pallas as pl
from jax.experimental.pallas import tpu as pltpu
from jax.experimental.pallas import tpu_sc as plsc

N = 10000
NP = 10240
E = 160000
NW = 32
CH = 128
SLABS = 40
EP = NW * SLABS * CH
ROWS_PER_TILE = NP // 16
BM = 512


def _mesh():
    return plsc.VectorSubcoreMesh(core_axis_name="c", subcore_axis_name="s")


def _deg_body(dst_hbm, ones_hbm, zeros_hbm, out_hbm, idx_v, ones_v, dsem,
              acc_sh):
    c = lax.axis_index("c")
    s = lax.axis_index("s")
    w = c * 16 + s
    base = s * ROWS_PER_TILE
    pltpu.sync_copy(zeros_hbm, acc_sh.at[pl.ds(base, ROWS_PER_TILE)])
    pltpu.sync_copy(dst_hbm.at[w], idx_v)
    pltpu.sync_copy(ones_hbm, ones_v)
    plsc.subcore_barrier()

    def body(g, carry):
        descs = [
            pltpu.async_copy(ones_v, acc_sh.at[idx_v.at[g * 8 + b]],
                             dsem, add=True)
            for b in range(8)
        ]
        for d in descs:
            d.wait()
        return carry

    lax.fori_loop(0, SLABS // 8, body, 0)
    plsc.subcore_barrier()
    pltpu.sync_copy(acc_sh.at[pl.ds(base, ROWS_PER_TILE)],
                    out_hbm.at[c, pl.ds(base, ROWS_PER_TILE)])


def _deg_call(dst_r, ones128, zeros128):
    k = functools.partial(
        pl.kernel,
        mesh=_mesh(),
        out_type=jax.ShapeDtypeStruct((2, NP, 128), jnp.float32),
        scratch_types=[
            pltpu.VMEM((SLABS, CH), jnp.int32),
            pltpu.VMEM((CH, 128), jnp.float32),
            pltpu.SemaphoreType.DMA,
            pltpu.VMEM_SHARED((NP, 128), jnp.float32),
        ],
    )(_deg_body)
    return k(dst_r, ones128, zeros128)


NBUF = 2
NCHUNK = EP // CH
SA = 64
SB = 80 - SA
FAST_C = 1
ACC_RPT = 632
ACC_ROWS = 16 * ACC_RPT


def _prop_body(g_hbm, src_hbm, dst_hbm, zeros_hbm, out_hbm,
               sidx_v, didx_v, rows_v, gsem, ssem, acc_sh):
    c = lax.axis_index("c")
    s = lax.axis_index("s")
    base = s * ACC_RPT
    pltpu.sync_copy(zeros_hbm, acc_sh.at[pl.ds(base, ACC_RPT)])

    def run(start, n_chunks):
        pltpu.sync_copy(src_hbm.at[pl.ds(start, n_chunks)],
                        sidx_v.at[pl.ds(0, n_chunks)])
        pltpu.sync_copy(dst_hbm.at[pl.ds(start, n_chunks)],
                        didx_v.at[pl.ds(0, n_chunks)])
        plsc.subcore_barrier()

        def body(g, carry):
            gd = [
                pltpu.async_copy(g_hbm.at[sidx_v.at[g * NBUF + b]],
                                 rows_v.at[b], gsem.at[b])
                for b in range(NBUF)
            ]
            sd = []
            for b in range(NBUF):
                gd[b].wait()
                sd.append(pltpu.async_copy(
                    rows_v.at[b], acc_sh.at[didx_v.at[g * NBUF + b]],
                    ssem.at[b], add=True))
            for d in sd:
                d.wait()
            return carry

        lax.fori_loop(0, n_chunks // NBUF, body, 0)

    @pl.when(c == FAST_C)
    def _():
        run(s * SA, SA)

    @pl.when(c != FAST_C)
    def _():
        run(16 * SA + s * SB, SB)

    plsc.subcore_barrier()
    pltpu.sync_copy(acc_sh.at[pl.ds(base, ACC_RPT)],
                    out_hbm.at[c, pl.ds(base, ACC_RPT)])


def _prop_call(g, src_f, dst_f, zeros128):
    k = functools.partial(
        pl.kernel,
        mesh=_mesh(),
        out_type=jax.ShapeDtypeStruct((2, ACC_ROWS, 128), jnp.float32),
        scratch_types=[
            pltpu.VMEM((SA, CH), jnp.int32),
            pltpu.VMEM((SA, CH), jnp.int32),
            pltpu.VMEM((NBUF, CH, 128), jnp.float32),
            pltpu.SemaphoreType.DMA((NBUF,)),
            pltpu.SemaphoreType.DMA((NBUF,)),
            pltpu.VMEM_SHARED((ACC_ROWS, 128), jnp.float32),
        ],
    )(_prop_body)
    return k(g, src_f, dst_f, zeros128)


def _mm_body(x_ref, w_ref, o_ref):
    o_ref[...] = jnp.dot(x_ref[...], w_ref[...],
                         preferred_element_type=jnp.float32)


def _mm_call(x_pad, W1):
    return pl.pallas_call(
        _mm_body,
        grid=(NP // BM,),
        in_specs=[
            pl.BlockSpec((BM, 128), lambda i: (i, 0)),
            pl.BlockSpec((128, 128), lambda i: (0, 0)),
        ],
        out_specs=pl.BlockSpec((BM, 128), lambda i: (i, 0)),
        out_shape=jax.ShapeDtypeStruct((NP, 128), jnp.float32),
    )(x_pad, W1)


def _enc_a_body(h1_ref, degp_ref, g1_ref, dinv_ref):
    deg = degp_ref[0, :, :16] + degp_ref[1, :, :16] + 1.0
    dinv = lax.rsqrt(deg)
    dinv_ref[...] = dinv
    g1_ref[...] = h1_ref[...] * dinv[:, 0:1]


def _enc_a_call(h1, degp):
    return pl.pallas_call(
        _enc_a_body,
        grid=(NP // BM,),
        in_specs=[
            pl.BlockSpec((BM, 128), lambda i: (i, 0)),
            pl.BlockSpec((2, BM, 128), lambda i: (0, i, 0)),
        ],
        out_specs=[
            pl.BlockSpec((BM, 128), lambda i: (i, 0)),
            pl.BlockSpec((BM, 16), lambda i: (i, 0)),
        ],
        out_shape=[
            jax.ShapeDtypeStruct((NP, 128), jnp.float32),
            jax.ShapeDtypeStruct((NP, 16), jnp.float32),
        ],
    )(h1, degp)


def _enc_b_body(s1p_ref, g1_ref, dinv_ref, b1_ref, wc_ref, g2_ref):
    dinv = dinv_ref[...][:, 0:1]
    h = s1p_ref[0] + s1p_ref[1] + g1_ref[...]
    h = jnp.maximum(h * dinv + b1_ref[...], 0.0)
    c2 = jnp.dot(h, wc_ref[...], preferred_element_type=jnp.float32)
    g2_ref[...] = c2 * dinv


def _enc_b_call(s1p, g1, dinv16, b1r, Wc):
    return pl.pallas_call(
        _enc_b_body,
        grid=(NP // BM,),
        in_specs=[
            pl.BlockSpec((2, BM, 128), lambda i: (0, i, 0)),
            pl.BlockSpec((BM, 128), lambda i: (i, 0)),
            pl.BlockSpec((BM, 16), lambda i: (i, 0)),
            pl.BlockSpec((1, 128), lambda i: (0, 0)),
            pl.BlockSpec((128, 128), lambda i: (0, 0)),
        ],
        out_specs=pl.BlockSpec((BM, 128), lambda i: (i, 0)),
        out_shape=jax.ShapeDtypeStruct((NP, 128), jnp.float32),
    )(s1p, g1, dinv16, b1r, Wc)


def _enc_c_body(s2p_ref, g2_ref, dinv_ref, bc_ref, out_ref):
    dinv = dinv_ref[...][:, 0:1]
    t = s2p_ref[0] + s2p_ref[1] + g2_ref[...]
    out_ref[...] = t * dinv + bc_ref[...]


def _enc_c_call(s2p, g2, dinv16, bcr):
    return pl.pallas_call(
        _enc_c_body,
        grid=(NP // BM,),
        in_specs=[
            pl.BlockSpec((2, BM, 128), lambda i: (0, i, 0)),
            pl.BlockSpec((BM, 128), lambda i: (i, 0)),
            pl.BlockSpec((BM, 16), lambda i: (i, 0)),
            pl.BlockSpec((1, 128), lambda i: (0, 0)),
        ],
        out_specs=pl.BlockSpec((BM, 128), lambda i: (i, 0)),
        out_shape=jax.ShapeDtypeStruct((NP, 128), jnp.float32),
    )(s2p, g2, dinv16, bcr)


DM = 2048
DN = 2048


def _dec_body(a_ref, b_ref, o_ref):
    acc = lax.dot_general(a_ref[...], b_ref[...],
                          (((1,), (1,)), ((), ())),
                          preferred_element_type=jnp.float32)
    o_ref[...] = jax.nn.sigmoid(acc)


def _dec_call(z64):
    return pl.pallas_call(
        _dec_body,
        grid=(NP // DM, NP // DN),
        in_specs=[
            pl.BlockSpec((DM, 64), lambda i, j: (i, 0)),
            pl.BlockSpec((DN, 64), lambda i, j: (j, 0)),
        ],
        out_specs=pl.BlockSpec((DM, DN), lambda i, j: (i, j)),
        out_shape=jax.ShapeDtypeStruct((N, N), jnp.float32),
    )(z64, z64)


def kernel(x, edge_index, W1, b1, Wmu, bmu, Wls, bls):
    src = edge_index[0].astype(jnp.int32)
    dst = edge_index[1].astype(jnp.int32)
    pad = jnp.full((EP - E,), N, jnp.int32)
    src_f = jnp.concatenate([src, pad]).reshape(NCHUNK, CH)
    dst_f = jnp.concatenate([dst, pad]).reshape(NCHUNK, CH)
    dst_r = dst_f.reshape(NW, SLABS, CH)
    x_pad = jnp.pad(x, ((0, NP - N), (0, 0)))
    Wc = jnp.concatenate([Wmu, Wls], axis=1)
    bc = jnp.concatenate([bmu, bls]).reshape(1, 128)
    b1r = b1.reshape(1, 128)
    ones128 = jnp.ones((CH, 128), jnp.float32)
    zeros128 = jnp.zeros((ROWS_PER_TILE, 128), jnp.float32)
    zeros632 = jnp.zeros((ACC_RPT, 128), jnp.float32)

    degp = _deg_call(dst_r, ones128, zeros128)
    h1 = _mm_call(x_pad, W1)
    g1, dinv16 = _enc_a_call(h1, degp)
    s1p = _prop_call(g1, src_f, dst_f, zeros632)
    g2 = _enc_b_call(s1p, g1, dinv16, b1r, Wc)
    s2p = _prop_call(g2, src_f, dst_f, zeros632)
    full2 = _enc_c_call(s2p, g2, dinv16, bc)

    mu = full2[:N, :64]
    logstd = full2[:N, 64:]
    adj = _dec_call(full2[:, :64])
    return adj, mu, logstd


# speedup vs baseline: 8.0327x; 1.0022x over previous
import functools

import jax
import jax.numpy as jnp
from jax import lax
from jax.experimental import pallas as pl
from jax.experimental.pallas import tpu as pltpu
from jax.experimental.pallas import tpu_sc as plsc

N = 10000
NP = 10240
E = 160000
NW = 32
CH = 128
SLABS = 40
EP = NW * SLABS * CH
ROWS_PER_TILE = NP // 16
BM = 512


def _mesh():
    return plsc.VectorSubcoreMesh(core_axis_name="c", subcore_axis_name="s")


def _deg_body(dst_hbm, ones_hbm, zeros_hbm, out_hbm, idx_v, ones_v, dsem,
              acc_sh):
    c = lax.axis_index("c")
    s = lax.axis_index("s")
    w = c * 16 + s
    base = s * ROWS_PER_TILE
    pltpu.sync_copy(zeros_hbm, acc_sh.at[pl.ds(base, ROWS_PER_TILE)])
    pltpu.sync_copy(dst_hbm.at[w], idx_v)
    pltpu.sync_copy(ones_hbm, ones_v)
    plsc.subcore_barrier()

    def body(g, carry):
        descs = [
            pltpu.async_copy(ones_v, acc_sh.at[idx_v.at[g * 8 + b]],
                             dsem, add=True)
            for b in range(8)
        ]
        for d in descs:
            d.wait()
        return carry

    lax.fori_loop(0, SLABS // 8, body, 0)
    plsc.subcore_barrier()
    pltpu.sync_copy(acc_sh.at[pl.ds(base, ROWS_PER_TILE)],
                    out_hbm.at[c, pl.ds(base, ROWS_PER_TILE)])


def _deg_call(dst_r, ones128, zeros128):
    k = functools.partial(
        pl.kernel,
        mesh=_mesh(),
        out_type=jax.ShapeDtypeStruct((2, NP, 128), jnp.float32),
        scratch_types=[
            pltpu.VMEM((SLABS, CH), jnp.int32),
            pltpu.VMEM((CH, 128), jnp.float32),
            pltpu.SemaphoreType.DMA,
            pltpu.VMEM_SHARED((NP, 128), jnp.float32),
        ],
    )(_deg_body)
    return k(dst_r, ones128, zeros128)


NBUF = 2
NCHUNK = EP // CH
SA = 64
SB = 80 - SA
FAST_C = 1
ACC_RPT = 632
ACC_ROWS = 16 * ACC_RPT


def _prop_body(g_hbm, src_hbm, dst_hbm, zeros_hbm, out_hbm,
               sidx_v, didx_v, rows_v, gsem, ssem, acc_sh):
    c = lax.axis_index("c")
    s = lax.axis_index("s")
    base = s * ACC_RPT
    pltpu.sync_copy(zeros_hbm, acc_sh.at[pl.ds(base, ACC_RPT)])

    def run(start, n_chunks):
        pltpu.sync_copy(src_hbm.at[pl.ds(start, n_chunks)],
                        sidx_v.at[pl.ds(0, n_chunks)])
        pltpu.sync_copy(dst_hbm.at[pl.ds(start, n_chunks)],
                        didx_v.at[pl.ds(0, n_chunks)])
        plsc.subcore_barrier()

        def body(g, carry):
            gd = [
                pltpu.async_copy(g_hbm.at[sidx_v.at[g * NBUF + b]],
                                 rows_v.at[b], gsem.at[b])
                for b in range(NBUF)
            ]
            sd = []
            for b in range(NBUF):
                gd[b].wait()
                sd.append(pltpu.async_copy(
                    rows_v.at[b], acc_sh.at[didx_v.at[g * NBUF + b]],
                    ssem.at[b], add=True))
            for d in sd:
                d.wait()
            return carry

        lax.fori_loop(0, n_chunks // NBUF, body, 0)

    @pl.when(c == FAST_C)
    def _():
        run(s * SA, SA)

    @pl.when(c != FAST_C)
    def _():
        run(16 * SA + s * SB, SB)

    plsc.subcore_barrier()
    pltpu.sync_copy(acc_sh.at[pl.ds(base, ACC_RPT)],
                    out_hbm.at[c, pl.ds(base, ACC_RPT)])


def _prop_call(g, src_f, dst_f, zeros128):
    k = functools.partial(
        pl.kernel,
        mesh=_mesh(),
        out_type=jax.ShapeDtypeStruct((2, ACC_ROWS, 128), jnp.float32),
        scratch_types=[
            pltpu.VMEM((SA, CH), jnp.int32),
            pltpu.VMEM((SA, CH), jnp.int32),
            pltpu.VMEM((NBUF, CH, 128), jnp.float32),
            pltpu.SemaphoreType.DMA((NBUF,)),
            pltpu.SemaphoreType.DMA((NBUF,)),
            pltpu.VMEM_SHARED((ACC_ROWS, 128), jnp.float32),
        ],
    )(_prop_body)
    return k(g, src_f, dst_f, zeros128)


def _mm_body(x_ref, w_ref, o_ref):
    o_ref[...] = jnp.dot(x_ref[...], w_ref[...],
                         preferred_element_type=jnp.float32)


def _mm_call(x_pad, W1):
    return pl.pallas_call(
        _mm_body,
        grid=(NP // BM,),
        in_specs=[
            pl.BlockSpec((BM, 128), lambda i: (i, 0)),
            pl.BlockSpec((128, 128), lambda i: (0, 0)),
        ],
        out_specs=pl.BlockSpec((BM, 128), lambda i: (i, 0)),
        out_shape=jax.ShapeDtypeStruct((NP, 128), jnp.float32),
    )(x_pad, W1)


def _enc_a_body(h1_ref, degp_ref, g1_ref, dinv_ref):
    deg = degp_ref[0, :, :16] + degp_ref[1, :, :16] + 1.0
    dinv = lax.rsqrt(deg)
    dinv_ref[...] = dinv
    g1_ref[...] = h1_ref[...] * dinv[:, 0:1]


def _enc_a_call(h1, degp):
    return pl.pallas_call(
        _enc_a_body,
        grid=(NP // BM,),
        in_specs=[
            pl.BlockSpec((BM, 128), lambda i: (i, 0)),
            pl.BlockSpec((2, BM, 128), lambda i: (0, i, 0)),
        ],
        out_specs=[
            pl.BlockSpec((BM, 128), lambda i: (i, 0)),
            pl.BlockSpec((BM, 16), lambda i: (i, 0)),
        ],
        out_shape=[
            jax.ShapeDtypeStruct((NP, 128), jnp.float32),
            jax.ShapeDtypeStruct((NP, 16), jnp.float32),
        ],
    )(h1, degp)


def _enc_b_body(s1p_ref, g1_ref, dinv_ref, b1_ref, wc_ref, g2_ref):
    dinv = dinv_ref[...][:, 0:1]
    h = s1p_ref[0] + s1p_ref[1] + g1_ref[...]
    h = jnp.maximum(h * dinv + b1_ref[...], 0.0)
    c2 = jnp.dot(h, wc_ref[...], preferred_element_type=jnp.float32)
    g2_ref[...] = c2 * dinv


def _enc_b_call(s1p, g1, dinv16, b1r, Wc):
    return pl.pallas_call(
        _enc_b_body,
        grid=(NP // BM,),
        in_specs=[
            pl.BlockSpec((2, BM, 128), lambda i: (0, i, 0)),
            pl.BlockSpec((BM, 128), lambda i: (i, 0)),
            pl.BlockSpec((BM, 16), lambda i: (i, 0)),
            pl.BlockSpec((1, 128), lambda i: (0, 0)),
            pl.BlockSpec((128, 128), lambda i: (0, 0)),
        ],
        out_specs=pl.BlockSpec((BM, 128), lambda i: (i, 0)),
        out_shape=jax.ShapeDtypeStruct((NP, 128), jnp.float32),
    )(s1p, g1, dinv16, b1r, Wc)


def _enc_c_body(s2p_ref, g2_ref, dinv_ref, bc_ref, out_ref):
    dinv = dinv_ref[...][:, 0:1]
    t = s2p_ref[0] + s2p_ref[1] + g2_ref[...]
    out_ref[...] = t * dinv + bc_ref[...]


def _enc_c_call(s2p, g2, dinv16, bcr):
    return pl.pallas_call(
        _enc_c_body,
        grid=(NP // BM,),
        in_specs=[
            pl.BlockSpec((2, BM, 128), lambda i: (0, i, 0)),
            pl.BlockSpec((BM, 128), lambda i: (i, 0)),
            pl.BlockSpec((BM, 16), lambda i: (i, 0)),
            pl.BlockSpec((1, 128), lambda i: (0, 0)),
        ],
        out_specs=pl.BlockSpec((BM, 128), lambda i: (i, 0)),
        out_shape=jax.ShapeDtypeStruct((NP, 128), jnp.float32),
    )(s2p, g2, dinv16, bcr)


DM = 2560
DN = 2048


def _dec_body(a_ref, b_ref, o_ref):
    acc = lax.dot_general(a_ref[...], b_ref[...],
                          (((1,), (1,)), ((), ())),
                          preferred_element_type=jnp.float32)
    o_ref[...] = jax.nn.sigmoid(acc)


def _dec_call(z64):
    return pl.pallas_call(
        _dec_body,
        grid=(NP // DM, NP // DN),
        in_specs=[
            pl.BlockSpec((DM, 64), lambda i, j: (i, 0)),
            pl.BlockSpec((DN, 64), lambda i, j: (j, 0)),
        ],
        out_specs=pl.BlockSpec((DM, DN), lambda i, j: (i, j)),
        out_shape=jax.ShapeDtypeStruct((N, N), jnp.float32),
    )(z64, z64)


def kernel(x, edge_index, W1, b1, Wmu, bmu, Wls, bls):
    src = edge_index[0].astype(jnp.int32)
    dst = edge_index[1].astype(jnp.int32)
    pad = jnp.full((EP - E,), N, jnp.int32)
    src_f = jnp.concatenate([src, pad]).reshape(NCHUNK, CH)
    dst_f = jnp.concatenate([dst, pad]).reshape(NCHUNK, CH)
    dst_r = dst_f.reshape(NW, SLABS, CH)
    x_pad = jnp.pad(x, ((0, NP - N), (0, 0)))
    Wc = jnp.concatenate([Wmu, Wls], axis=1)
    bc = jnp.concatenate([bmu, bls]).reshape(1, 128)
    b1r = b1.reshape(1, 128)
    ones128 = jnp.ones((CH, 128), jnp.float32)
    zeros128 = jnp.zeros((ROWS_PER_TILE, 128), jnp.float32)
    zeros632 = jnp.zeros((ACC_RPT, 128), jnp.float32)

    degp = _deg_call(dst_r, ones128, zeros128)
    h1 = _mm_call(x_pad, W1)
    g1, dinv16 = _enc_a_call(h1, degp)
    s1p = _prop_call(g1, src_f, dst_f, zeros632)
    g2 = _enc_b_call(s1p, g1, dinv16, b1r, Wc)
    s2p = _prop_call(g2, src_f, dst_f, zeros632)
    full2 = _enc_c_call(s2p, g2, dinv16, bc)

    mu = full2[:N, :64]
    logstd = full2[:N, 64:]
    adj = _dec_call(full2[:, :64])
    return adj, mu, logstd
